# Initial kernel scaffold; baseline (speedup 1.0000x reference)
import jax
import jax.numpy as jnp
from jax.experimental import pallas as pl


def kernel(edge_index, x_list, ix, aug_loss, W1, b1, W2, b2):
    raise NotImplementedError("write your pallas kernel here")


# speedup vs baseline: 8.9000x; 8.9000x over previous
import functools

import jax
import jax.numpy as jnp
from jax import lax
from jax.experimental import pallas as pl
from jax.experimental.pallas import tpu as pltpu
from jax.experimental.pallas import tpu_sc as plsc

NC = 2
NS = 16
NW = NC * NS
BLK = 128


def _deg_kernel(np_, d, epw, nblk):
  rps = np_ // NS
  mesh = plsc.VectorSubcoreMesh(core_axis_name="c", subcore_axis_name="s")

  def body(dst_hbm, ones_hbm, zeros_hbm, out_hbm, idx_v, ones_v, acc_sh):
    cid = lax.axis_index("c")
    sid = lax.axis_index("s")
    wid = cid * NS + sid
    pltpu.sync_copy(zeros_hbm, acc_sh.at[pl.ds(sid * rps, rps)])
    pltpu.sync_copy(ones_hbm, ones_v)
    plsc.subcore_barrier()

    @pl.loop(0, nblk)
    def _(j):
      pltpu.sync_copy(dst_hbm.at[pl.ds(wid * epw + j * BLK, BLK)], idx_v)
      pltpu.sync_copy(ones_v, acc_sh.at[idx_v], add=True)

    plsc.subcore_barrier()
    pltpu.sync_copy(acc_sh.at[pl.ds(sid * rps, rps)],
                    out_hbm.at[pl.ds(cid * np_ + sid * rps, rps)])

  return pl.kernel(
      body, mesh=mesh,
      out_type=jax.ShapeDtypeStruct((NC * np_, d), jnp.float32),
      scratch_types=[
          pltpu.VMEM((BLK,), jnp.int32),
          pltpu.VMEM((BLK, d), jnp.float32),
          pltpu.VMEM_SHARED((np_, d), jnp.float32),
      ])


def _edge_scatter_kernel(np_, d, epw, nblk):
  rps = np_ // NS
  mesh = plsc.VectorSubcoreMesh(core_axis_name="c", subcore_axis_name="s")

  def body(g_hbm, src_hbm, dst_hbm, zeros_hbm, out_hbm, idx_s, idx_d, rows_v,
           acc_sh):
    cid = lax.axis_index("c")
    sid = lax.axis_index("s")
    wid = cid * NS + sid
    pltpu.sync_copy(zeros_hbm, acc_sh.at[pl.ds(sid * rps, rps)])
    plsc.subcore_barrier()

    @pl.loop(0, nblk)
    def _(j):
      base = wid * epw + j * BLK
      pltpu.sync_copy(src_hbm.at[pl.ds(base, BLK)], idx_s)
      pltpu.sync_copy(dst_hbm.at[pl.ds(base, BLK)], idx_d)
      pltpu.sync_copy(g_hbm.at[idx_s], rows_v)
      pltpu.sync_copy(rows_v, acc_sh.at[idx_d], add=True)

    plsc.subcore_barrier()
    pltpu.sync_copy(acc_sh.at[pl.ds(sid * rps, rps)],
                    out_hbm.at[pl.ds(cid * np_ + sid * rps, rps)])

  return pl.kernel(
      body, mesh=mesh,
      out_type=jax.ShapeDtypeStruct((NC * np_, d), jnp.float32),
      scratch_types=[
          pltpu.VMEM((BLK,), jnp.int32),
          pltpu.VMEM((BLK,), jnp.int32),
          pltpu.VMEM((BLK, d), jnp.float32),
          pltpu.VMEM_SHARED((np_, d), jnp.float32),
      ])


def _b_body(d0_ref, d1_ref, x_ref, g0_ref, dinv_ref):
  deg = d0_ref[:, 0:1] + d1_ref[:, 0:1] + 1.0
  dinv = lax.rsqrt(deg)
  db = jnp.broadcast_to(dinv, x_ref.shape)
  dinv_ref[...] = db
  g0_ref[...] = db * x_ref[...]


def _d_body(s0a_ref, s0b_ref, x_ref, dinv_ref, w1_ref, b1_ref, w2_ref,
            g1_ref, u_ref):
  dinv = dinv_ref[...]
  y0 = dinv * (s0a_ref[...] + s0b_ref[...]) + dinv * dinv * x_ref[...]
  h = jnp.dot(y0, w1_ref[...], preferred_element_type=jnp.float32,
              precision=lax.Precision.HIGHEST) + b1_ref[...]
  z = jnp.maximum(h, 0.0)
  u = jnp.dot(z, w2_ref[...], preferred_element_type=jnp.float32,
              precision=lax.Precision.HIGHEST)
  u_ref[...] = u
  g1_ref[...] = dinv * u


def _f_body(s1a_ref, s1b_ref, u_ref, dinv_ref, b2_ref, out_ref):
  dinv = dinv_ref[...]
  v = dinv * (s1a_ref[...] + s1b_ref[...]) + dinv * dinv * u_ref[...]
  v = v + b2_ref[...]
  d = v.shape[1]
  dd = d // 4
  outs = []
  for k in range(4):
    vk = v[:, k * dd:(k + 1) * dd]
    n2 = jnp.sum(vk * vk, axis=1, keepdims=True)
    nr = jnp.maximum(jnp.sqrt(n2), 1e-12)
    outs.append(vk / nr)
  out_ref[...] = jnp.concatenate(outs, axis=1)


def kernel(edge_index, x_list, ix, aug_loss, W1, b1, W2, b2):
  n, d = x_list.shape
  e = edge_index.shape[1]
  d4 = W1.shape[1]

  np_ = ((n + 1279) // 1280) * 1280
  epw = ((e + NW * BLK - 1) // (NW * BLK)) * BLK
  ep = epw * NW
  nblk = epw // BLK

  src = jnp.concatenate(
      [edge_index[0], jnp.full((ep - e,), np_ - 1, jnp.int32)])
  dst = jnp.concatenate(
      [edge_index[1], jnp.full((ep - e,), np_ - 1, jnp.int32)])
  xp = jnp.pad(x_list, ((0, np_ - n), (0, 0)))
  onesd = jnp.ones((BLK, d), jnp.float32)
  zerosd = jnp.zeros((np_ // NS, d), jnp.float32)
  b1r = b1.reshape(1, d4)
  b2r = b2.reshape(1, d)

  degp = _deg_kernel(np_, d, epw, nblk)(dst, onesd, zerosd)
  d0 = degp[:np_]
  d1 = degp[np_:]

  rb = 1024
  grid = (np_ // rb,)
  g0, dinvb = pl.pallas_call(
      _b_body,
      grid=grid,
      in_specs=[
          pl.BlockSpec((rb, d), lambda i: (i, 0)),
          pl.BlockSpec((rb, d), lambda i: (i, 0)),
          pl.BlockSpec((rb, d), lambda i: (i, 0)),
      ],
      out_specs=[
          pl.BlockSpec((rb, d), lambda i: (i, 0)),
          pl.BlockSpec((rb, d), lambda i: (i, 0)),
      ],
      out_shape=[
          jax.ShapeDtypeStruct((np_, d), jnp.float32),
          jax.ShapeDtypeStruct((np_, d), jnp.float32),
      ],
  )(d0, d1, xp)

  scat = _edge_scatter_kernel(np_, d, epw, nblk)
  s0 = scat(g0, src, dst, zerosd)

  g1, u = pl.pallas_call(
      _d_body,
      grid=grid,
      in_specs=[
          pl.BlockSpec((rb, d), lambda i: (i, 0)),
          pl.BlockSpec((rb, d), lambda i: (i, 0)),
          pl.BlockSpec((rb, d), lambda i: (i, 0)),
          pl.BlockSpec((rb, d), lambda i: (i, 0)),
          pl.BlockSpec((d, d4), lambda i: (0, 0)),
          pl.BlockSpec((1, d4), lambda i: (0, 0)),
          pl.BlockSpec((d4, d), lambda i: (0, 0)),
      ],
      out_specs=[
          pl.BlockSpec((rb, d), lambda i: (i, 0)),
          pl.BlockSpec((rb, d), lambda i: (i, 0)),
      ],
      out_shape=[
          jax.ShapeDtypeStruct((np_, d), jnp.float32),
          jax.ShapeDtypeStruct((np_, d), jnp.float32),
      ],
  )(s0[:np_], s0[np_:], xp, dinvb, W1, b1r, W2)

  s1 = scat(g1, src, dst, zerosd)

  out = pl.pallas_call(
      _f_body,
      grid=grid,
      in_specs=[
          pl.BlockSpec((rb, d), lambda i: (i, 0)),
          pl.BlockSpec((rb, d), lambda i: (i, 0)),
          pl.BlockSpec((rb, d), lambda i: (i, 0)),
          pl.BlockSpec((rb, d), lambda i: (i, 0)),
          pl.BlockSpec((1, d), lambda i: (0, 0)),
      ],
      out_specs=pl.BlockSpec((rb, d), lambda i: (i, 0)),
      out_shape=jax.ShapeDtypeStruct((np_, d), jnp.float32),
  )(s1[:np_], s1[np_:], u, dinvb, b2r)

  return out[:n]


# speedup vs baseline: 10.5091x; 1.1808x over previous
import functools

import jax
import jax.numpy as jnp
from jax import lax
from jax.experimental import pallas as pl
from jax.experimental.pallas import tpu as pltpu
from jax.experimental.pallas import tpu_sc as plsc

NC = 2
NS = 16
NW = NC * NS
BLK = 128


def _deg_kernel(np_, d, epw, nblk):
  rps = np_ // NS
  mesh = plsc.VectorSubcoreMesh(core_axis_name="c", subcore_axis_name="s")

  def body(dst_hbm, ones_hbm, zeros_hbm, out_hbm, idx_d, ones_v, acc_sh):
    cid = lax.axis_index("c")
    sid = lax.axis_index("s")
    wid = cid * NS + sid
    pltpu.sync_copy(dst_hbm.at[pl.ds(wid * nblk, nblk)], idx_d)
    pltpu.sync_copy(ones_hbm, ones_v)
    pltpu.sync_copy(zeros_hbm, acc_sh.at[pl.ds(sid * rps, rps)])
    plsc.subcore_barrier()

    @pl.loop(0, nblk)
    def _(j):
      pltpu.sync_copy(ones_v, acc_sh.at[idx_d.at[j]], add=True)

    plsc.subcore_barrier()
    pltpu.sync_copy(acc_sh.at[pl.ds(sid * rps, rps)],
                    out_hbm.at[pl.ds(cid * np_ + sid * rps, rps)])

  return pl.kernel(
      body, mesh=mesh,
      out_type=jax.ShapeDtypeStruct((NC * np_, d), jnp.float32),
      scratch_types=[
          pltpu.VMEM((nblk, BLK), jnp.int32),
          pltpu.VMEM((BLK, d), jnp.float32),
          pltpu.VMEM_SHARED((np_, d), jnp.float32),
      ])


def _edge_scatter_kernel(np_, d, epw, nblk):
  rps = np_ // NS
  mesh = plsc.VectorSubcoreMesh(core_axis_name="c", subcore_axis_name="s")

  def body(g_hbm, src_hbm, dst_hbm, zeros_hbm, out_hbm, idx_s, idx_d,
           rows0, rows1, sem0, sem1, acc_sh):
    cid = lax.axis_index("c")
    sid = lax.axis_index("s")
    wid = cid * NS + sid
    pltpu.sync_copy(src_hbm.at[pl.ds(wid * nblk, nblk)], idx_s)
    pltpu.sync_copy(dst_hbm.at[pl.ds(wid * nblk, nblk)], idx_d)
    pltpu.async_copy(g_hbm.at[idx_s.at[0]], rows0, sem0)
    pltpu.async_copy(g_hbm.at[idx_s.at[1]], rows1, sem1)
    pltpu.sync_copy(zeros_hbm, acc_sh.at[pl.ds(sid * rps, rps)])
    plsc.subcore_barrier()

    @pl.loop(0, nblk // 2 - 1)
    def _(jj):
      j0 = 2 * jj
      pltpu.make_async_copy(g_hbm.at[idx_s.at[j0]], rows0, sem0).wait()
      pltpu.sync_copy(rows0, acc_sh.at[idx_d.at[j0]], add=True)
      pltpu.async_copy(g_hbm.at[idx_s.at[j0 + 2]], rows0, sem0)
      pltpu.make_async_copy(g_hbm.at[idx_s.at[j0 + 1]], rows1, sem1).wait()
      pltpu.sync_copy(rows1, acc_sh.at[idx_d.at[j0 + 1]], add=True)
      pltpu.async_copy(g_hbm.at[idx_s.at[j0 + 3]], rows1, sem1)

    pltpu.make_async_copy(g_hbm.at[idx_s.at[nblk - 2]], rows0, sem0).wait()
    pltpu.sync_copy(rows0, acc_sh.at[idx_d.at[nblk - 2]], add=True)
    pltpu.make_async_copy(g_hbm.at[idx_s.at[nblk - 1]], rows1, sem1).wait()
    pltpu.sync_copy(rows1, acc_sh.at[idx_d.at[nblk - 1]], add=True)

    plsc.subcore_barrier()
    pltpu.sync_copy(acc_sh.at[pl.ds(sid * rps, rps)],
                    out_hbm.at[pl.ds(cid * np_ + sid * rps, rps)])

  return pl.kernel(
      body, mesh=mesh,
      out_type=jax.ShapeDtypeStruct((NC * np_, d), jnp.float32),
      scratch_types=[
          pltpu.VMEM((nblk, BLK), jnp.int32),
          pltpu.VMEM((nblk, BLK), jnp.int32),
          pltpu.VMEM((BLK, d), jnp.float32),
          pltpu.VMEM((BLK, d), jnp.float32),
          pltpu.SemaphoreType.DMA,
          pltpu.SemaphoreType.DMA,
          pltpu.VMEM_SHARED((np_, d), jnp.float32),
      ])


def _b_body(d0_ref, d1_ref, x_ref, g0_ref, dinv_ref):
  deg = d0_ref[:, 0:1] + d1_ref[:, 0:1] + 1.0
  dinv = lax.rsqrt(deg)
  db = jnp.broadcast_to(dinv, x_ref.shape)
  dinv_ref[...] = db
  g0_ref[...] = db * x_ref[...]


def _d_body(s0a_ref, s0b_ref, x_ref, dinv_ref, w1_ref, b1_ref, w2_ref,
            g1_ref, u_ref):
  dinv = dinv_ref[...]
  y0 = dinv * (s0a_ref[...] + s0b_ref[...]) + dinv * dinv * x_ref[...]
  h = jnp.dot(y0, w1_ref[...], preferred_element_type=jnp.float32,
              precision=lax.Precision.HIGHEST) + b1_ref[...]
  z = jnp.maximum(h, 0.0)
  u = jnp.dot(z, w2_ref[...], preferred_element_type=jnp.float32,
              precision=lax.Precision.HIGHEST)
  u_ref[...] = u
  g1_ref[...] = dinv * u


def _f_body(s1a_ref, s1b_ref, u_ref, dinv_ref, b2_ref, out_ref):
  dinv = dinv_ref[...]
  v = dinv * (s1a_ref[...] + s1b_ref[...]) + dinv * dinv * u_ref[...]
  v = v + b2_ref[...]
  d = v.shape[1]
  dd = d // 4
  outs = []
  for k in range(4):
    vk = v[:, k * dd:(k + 1) * dd]
    n2 = jnp.sum(vk * vk, axis=1, keepdims=True)
    nr = jnp.maximum(jnp.sqrt(n2), 1e-12)
    outs.append(vk / nr)
  out_ref[...] = jnp.concatenate(outs, axis=1)


def kernel(edge_index, x_list, ix, aug_loss, W1, b1, W2, b2):
  n, d = x_list.shape
  e = edge_index.shape[1]
  d4 = W1.shape[1]

  np_ = ((n + 1279) // 1280) * 1280
  epw = ((e + NW * 2 * BLK - 1) // (NW * 2 * BLK)) * 2 * BLK
  ep = epw * NW
  nblk = epw // BLK

  src = jnp.concatenate(
      [edge_index[0], jnp.full((ep - e,), np_ - 1, jnp.int32)]
  ).reshape(ep // BLK, BLK)
  dst = jnp.concatenate(
      [edge_index[1], jnp.full((ep - e,), np_ - 1, jnp.int32)]
  ).reshape(ep // BLK, BLK)
  xp = jnp.pad(x_list, ((0, np_ - n), (0, 0)))
  onesd = jnp.ones((BLK, d), jnp.float32)
  zerosd = jnp.zeros((np_ // NS, d), jnp.float32)
  b1r = b1.reshape(1, d4)
  b2r = b2.reshape(1, d)

  degp = _deg_kernel(np_, d, epw, nblk)(dst, onesd, zerosd)
  d0 = degp[:np_]
  d1 = degp[np_:]

  rb = 1024
  grid = (np_ // rb,)
  g0, dinvb = pl.pallas_call(
      _b_body,
      grid=grid,
      in_specs=[
          pl.BlockSpec((rb, d), lambda i: (i, 0)),
          pl.BlockSpec((rb, d), lambda i: (i, 0)),
          pl.BlockSpec((rb, d), lambda i: (i, 0)),
      ],
      out_specs=[
          pl.BlockSpec((rb, d), lambda i: (i, 0)),
          pl.BlockSpec((rb, d), lambda i: (i, 0)),
      ],
      out_shape=[
          jax.ShapeDtypeStruct((np_, d), jnp.float32),
          jax.ShapeDtypeStruct((np_, d), jnp.float32),
      ],
  )(d0, d1, xp)

  scat = _edge_scatter_kernel(np_, d, epw, nblk)
  s0 = scat(g0, src, dst, zerosd)

  g1, u = pl.pallas_call(
      _d_body,
      grid=grid,
      in_specs=[
          pl.BlockSpec((rb, d), lambda i: (i, 0)),
          pl.BlockSpec((rb, d), lambda i: (i, 0)),
          pl.BlockSpec((rb, d), lambda i: (i, 0)),
          pl.BlockSpec((rb, d), lambda i: (i, 0)),
          pl.BlockSpec((d, d4), lambda i: (0, 0)),
          pl.BlockSpec((1, d4), lambda i: (0, 0)),
          pl.BlockSpec((d4, d), lambda i: (0, 0)),
      ],
      out_specs=[
          pl.BlockSpec((rb, d), lambda i: (i, 0)),
          pl.BlockSpec((rb, d), lambda i: (i, 0)),
      ],
      out_shape=[
          jax.ShapeDtypeStruct((np_, d), jnp.float32),
          jax.ShapeDtypeStruct((np_, d), jnp.float32),
      ],
  )(s0[:np_], s0[np_:], xp, dinvb, W1, b1r, W2)

  s1 = scat(g1, src, dst, zerosd)

  out = pl.pallas_call(
      _f_body,
      grid=grid,
      in_specs=[
          pl.BlockSpec((rb, d), lambda i: (i, 0)),
          pl.BlockSpec((rb, d), lambda i: (i, 0)),
          pl.BlockSpec((rb, d), lambda i: (i, 0)),
          pl.BlockSpec((rb, d), lambda i: (i, 0)),
          pl.BlockSpec((1, d), lambda i: (0, 0)),
      ],
      out_specs=pl.BlockSpec((rb, d), lambda i: (i, 0)),
      out_shape=jax.ShapeDtypeStruct((np_, d), jnp.float32),
  )(s1[:np_], s1[np_:], u, dinvb, b2r)

  return out[:n]


# speedup vs baseline: 10.9417x; 1.0412x over previous
import functools

import jax
import jax.numpy as jnp
from jax import lax
from jax.experimental import pallas as pl
from jax.experimental.pallas import tpu as pltpu
from jax.experimental.pallas import tpu_sc as plsc

NC = 2
NS = 16
NW = NC * NS
BLK = 128


def _deg_kernel(np_, d, epw, nblk):
  rps = np_ // NS
  mesh = plsc.VectorSubcoreMesh(core_axis_name="c", subcore_axis_name="s")

  def body(dst_hbm, ones_hbm, zeros_hbm, out_hbm, idx_d, ones_v, acc_sh):
    cid = lax.axis_index("c")
    sid = lax.axis_index("s")
    wid = cid * NS + sid
    pltpu.sync_copy(dst_hbm.at[pl.ds(wid * nblk, nblk)], idx_d)
    pltpu.sync_copy(ones_hbm, ones_v)
    pltpu.sync_copy(zeros_hbm, acc_sh.at[pl.ds(sid * rps, rps)])
    plsc.subcore_barrier()

    @pl.loop(0, nblk)
    def _(j):
      pltpu.sync_copy(ones_v, acc_sh.at[idx_d.at[j]], add=True)

    plsc.subcore_barrier()
    pltpu.sync_copy(acc_sh.at[pl.ds(sid * rps, rps)],
                    out_hbm.at[pl.ds(cid * np_ + sid * rps, rps)])

  return pl.kernel(
      body, mesh=mesh,
      out_type=jax.ShapeDtypeStruct((NC * np_, d), jnp.float32),
      scratch_types=[
          pltpu.VMEM((nblk, BLK), jnp.int32),
          pltpu.VMEM((BLK, d), jnp.float32),
          pltpu.VMEM_SHARED((np_, d), jnp.float32),
      ])


def _edge_scatter_kernel(np_, d, nb0, nb1):
  rps = np_ // NS
  nbmax = max(nb0, nb1)
  mesh = plsc.VectorSubcoreMesh(core_axis_name="c", subcore_axis_name="s")

  def body(g_hbm, src_hbm, dst_hbm, zeros_hbm, out_hbm, idx_s, idx_d,
           rows0, rows1, sem0, sem1, acc_sh):
    cid = lax.axis_index("c")
    sid = lax.axis_index("s")
    pltpu.sync_copy(zeros_hbm, acc_sh.at[pl.ds(sid * rps, rps)])
    plsc.subcore_barrier()

    def run(nblk_c, rowbase):
      pltpu.sync_copy(src_hbm.at[pl.ds(rowbase, nblk_c)],
                      idx_s.at[pl.ds(0, nblk_c)])
      pltpu.sync_copy(dst_hbm.at[pl.ds(rowbase, nblk_c)],
                      idx_d.at[pl.ds(0, nblk_c)])
      pltpu.async_copy(g_hbm.at[idx_s.at[0]], rows0, sem0)
      pltpu.async_copy(g_hbm.at[idx_s.at[1]], rows1, sem1)

      @pl.loop(0, nblk_c // 2 - 1)
      def _(jj):
        j0 = 2 * jj
        pltpu.make_async_copy(g_hbm.at[idx_s.at[j0]], rows0, sem0).wait()
        pltpu.sync_copy(rows0, acc_sh.at[idx_d.at[j0]], add=True)
        pltpu.async_copy(g_hbm.at[idx_s.at[j0 + 2]], rows0, sem0)
        pltpu.make_async_copy(g_hbm.at[idx_s.at[j0 + 1]], rows1, sem1).wait()
        pltpu.sync_copy(rows1, acc_sh.at[idx_d.at[j0 + 1]], add=True)
        pltpu.async_copy(g_hbm.at[idx_s.at[j0 + 3]], rows1, sem1)

      pltpu.make_async_copy(g_hbm.at[idx_s.at[nblk_c - 2]], rows0, sem0).wait()
      pltpu.sync_copy(rows0, acc_sh.at[idx_d.at[nblk_c - 2]], add=True)
      pltpu.make_async_copy(g_hbm.at[idx_s.at[nblk_c - 1]], rows1, sem1).wait()
      pltpu.sync_copy(rows1, acc_sh.at[idx_d.at[nblk_c - 1]], add=True)

    @pl.when(cid == 0)
    def _():
      run(nb0, sid * nb0)

    @pl.when(cid == 1)
    def _():
      run(nb1, NS * nb0 + sid * nb1)

    plsc.subcore_barrier()
    pltpu.sync_copy(acc_sh.at[pl.ds(sid * rps, rps)],
                    out_hbm.at[pl.ds(cid * np_ + sid * rps, rps)])

  return pl.kernel(
      body, mesh=mesh,
      out_type=jax.ShapeDtypeStruct((NC * np_, d), jnp.float32),
      scratch_types=[
          pltpu.VMEM((nbmax, BLK), jnp.int32),
          pltpu.VMEM((nbmax, BLK), jnp.int32),
          pltpu.VMEM((BLK, d), jnp.float32),
          pltpu.VMEM((BLK, d), jnp.float32),
          pltpu.SemaphoreType.DMA,
          pltpu.SemaphoreType.DMA,
          pltpu.VMEM_SHARED((np_, d), jnp.float32),
      ])


def _b_body(d0_ref, d1_ref, x_ref, g0_ref, dinv_ref):
  deg = d0_ref[:, 0:1] + d1_ref[:, 0:1] + 1.0
  dinv = lax.rsqrt(deg)
  db = jnp.broadcast_to(dinv, x_ref.shape)
  dinv_ref[...] = db
  g0_ref[...] = db * x_ref[...]


def _d_body(s0a_ref, s0b_ref, x_ref, dinv_ref, w1_ref, b1_ref, w2_ref,
            g1_ref, u_ref):
  dinv = dinv_ref[...]
  y0 = dinv * (s0a_ref[...] + s0b_ref[...]) + dinv * dinv * x_ref[...]
  h = jnp.dot(y0, w1_ref[...], preferred_element_type=jnp.float32,
              precision=lax.Precision.HIGHEST) + b1_ref[...]
  z = jnp.maximum(h, 0.0)
  u = jnp.dot(z, w2_ref[...], preferred_element_type=jnp.float32,
              precision=lax.Precision.HIGHEST)
  u_ref[...] = u
  g1_ref[...] = dinv * u


def _f_body(s1a_ref, s1b_ref, u_ref, dinv_ref, b2_ref, out_ref):
  dinv = dinv_ref[...]
  v = dinv * (s1a_ref[...] + s1b_ref[...]) + dinv * dinv * u_ref[...]
  v = v + b2_ref[...]
  d = v.shape[1]
  dd = d // 4
  outs = []
  for k in range(4):
    vk = v[:, k * dd:(k + 1) * dd]
    n2 = jnp.sum(vk * vk, axis=1, keepdims=True)
    nr = jnp.maximum(jnp.sqrt(n2), 1e-12)
    outs.append(vk / nr)
  out_ref[...] = jnp.concatenate(outs, axis=1)


def kernel(edge_index, x_list, ix, aug_loss, W1, b1, W2, b2):
  n, d = x_list.shape
  e = edge_index.shape[1]
  d4 = W1.shape[1]

  np_ = ((n + 1279) // 1280) * 1280
  epw = ((e + NW * 2 * BLK - 1) // (NW * 2 * BLK)) * 2 * BLK
  ep = epw * NW
  nblk = epw // BLK

  src = jnp.concatenate(
      [edge_index[0], jnp.full((ep - e,), np_ - 1, jnp.int32)]
  ).reshape(ep // BLK, BLK)
  dst = jnp.concatenate(
      [edge_index[1], jnp.full((ep - e,), np_ - 1, jnp.int32)]
  ).reshape(ep // BLK, BLK)
  xp = jnp.pad(x_list, ((0, np_ - n), (0, 0)))
  onesd = jnp.ones((BLK, d), jnp.float32)
  zerosd = jnp.zeros((np_ // NS, d), jnp.float32)
  b1r = b1.reshape(1, d4)
  b2r = b2.reshape(1, d)

  degp = _deg_kernel(np_, d, epw, nblk)(dst, onesd, zerosd)
  d0 = degp[:np_]
  d1 = degp[np_:]

  rb = 1024
  grid = (np_ // rb,)
  g0, dinvb = pl.pallas_call(
      _b_body,
      grid=grid,
      in_specs=[
          pl.BlockSpec((rb, d), lambda i: (i, 0)),
          pl.BlockSpec((rb, d), lambda i: (i, 0)),
          pl.BlockSpec((rb, d), lambda i: (i, 0)),
      ],
      out_specs=[
          pl.BlockSpec((rb, d), lambda i: (i, 0)),
          pl.BlockSpec((rb, d), lambda i: (i, 0)),
      ],
      out_shape=[
          jax.ShapeDtypeStruct((np_, d), jnp.float32),
          jax.ShapeDtypeStruct((np_, d), jnp.float32),
      ],
  )(d0, d1, xp)

  nbt = (ep // BLK) // NS
  nb0 = (nbt * 4 // 5) & ~1
  nb1 = nbt - nb0
  scat = _edge_scatter_kernel(np_, d, nb0, nb1)
  s0 = scat(g0, src, dst, zerosd)

  g1, u = pl.pallas_call(
      _d_body,
      grid=grid,
      in_specs=[
          pl.BlockSpec((rb, d), lambda i: (i, 0)),
          pl.BlockSpec((rb, d), lambda i: (i, 0)),
          pl.BlockSpec((rb, d), lambda i: (i, 0)),
          pl.BlockSpec((rb, d), lambda i: (i, 0)),
          pl.BlockSpec((d, d4), lambda i: (0, 0)),
          pl.BlockSpec((1, d4), lambda i: (0, 0)),
          pl.BlockSpec((d4, d), lambda i: (0, 0)),
      ],
      out_specs=[
          pl.BlockSpec((rb, d), lambda i: (i, 0)),
          pl.BlockSpec((rb, d), lambda i: (i, 0)),
      ],
      out_shape=[
          jax.ShapeDtypeStruct((np_, d), jnp.float32),
          jax.ShapeDtypeStruct((np_, d), jnp.float32),
      ],
  )(s0[:np_], s0[np_:], xp, dinvb, W1, b1r, W2)

  s1 = scat(g1, src, dst, zerosd)

  out = pl.pallas_call(
      _f_body,
      grid=grid,
      in_specs=[
          pl.BlockSpec((rb, d), lambda i: (i, 0)),
          pl.BlockSpec((rb, d), lambda i: (i, 0)),
          pl.BlockSpec((rb, d), lambda i: (i, 0)),
          pl.BlockSpec((rb, d), lambda i: (i, 0)),
          pl.BlockSpec((1, d), lambda i: (0, 0)),
      ],
      out_specs=pl.BlockSpec((rb, d), lambda i: (i, 0)),
      out_shape=jax.ShapeDtypeStruct((np_, d), jnp.float32),
  )(s1[:np_], s1[np_:], u, dinvb, b2r)

  return out[:n]
